# baseline (speedup 1.0000x reference)
import functools

import jax
import jax.numpy as jnp
from jax import lax
from jax.experimental import pallas as pl
from jax.experimental.pallas import tpu as pltpu
from jax.experimental.pallas import tpu_sc as plsc

D_MODEL = 64
SAMPLE = 200
GATHER_CHUNK = 100
N_CHUNKS = SAMPLE // GATHER_CHUNK
N_GROUPS = 13
TAIL_OUT = SAMPLE - 16
NBUF = 4


def _make_kernel(batch, n_per_worker):
    mesh = plsc.VectorSubcoreMesh(core_axis_name="c", subcore_axis_name="s")
    num_cores = 2

    @functools.partial(
        pl.kernel,
        out_type=jax.ShapeDtypeStruct((batch, 1, SAMPLE), jnp.float32),
        mesh=mesh,
        compiler_params=pltpu.CompilerParams(
            needs_layout_passes=False, use_tc_tiling_on_sc=False),
        scratch_types=[
            pltpu.VMEM((NBUF, N_CHUNKS, GATHER_CHUNK), jnp.int32),
            pltpu.VMEM((NBUF, SAMPLE, D_MODEL), jnp.float32),
            pltpu.VMEM((NBUF, D_MODEL), jnp.float32),
            pltpu.VMEM((NBUF, SAMPLE), jnp.float32),
            pltpu.SemaphoreType.DMA((NBUF,)),
            pltpu.SemaphoreType.DMA((NBUF,)),
            pltpu.SemaphoreType.DMA((NBUF,)),
            pltpu.SemaphoreType.DMA((NBUF,)),
        ],
    )
    def emb_dot(h_hbm, idx_hbm, w_hbm, out_hbm, idx_v, rows_v, h_v, out_v,
                idx_sems, rows_sems, h_sems, out_sems):
        wid = lax.axis_index("s") * num_cores + lax.axis_index("c")
        base_b = wid * n_per_worker

        lane = lax.iota(jnp.int32, 16)
        perm = {s: lane ^ s for s in (1, 2, 4, 8)}
        odd = {s: (lane & s) != 0 for s in (1, 2, 4, 8)}

        def issue_idx(k, slot):
            pltpu.async_copy(idx_hbm.at[base_b + k], idx_v.at[slot],
                             idx_sems.at[slot])

        def issue_rows(k, slot):
            for j in range(N_CHUNKS):
                pltpu.async_copy(
                    w_hbm.at[idx_v.at[slot, j]],
                    rows_v.at[slot, pl.ds(j * GATHER_CHUNK, GATHER_CHUNK)],
                    rows_sems.at[slot])

        def issue_h(k, slot):
            pltpu.async_copy(h_hbm.at[base_b + k, 0], h_v.at[slot],
                             h_sems.at[slot])

        def drain(dummy_hbm_src, dst_ref, sem):
            pltpu.make_async_copy(dummy_hbm_src, dst_ref, sem).wait()

        def shuffle(v, s):
            return jnp.take(v, perm[s], axis=0, unique_indices=True)

        def compute(slot):
            rows = rows_v.at[slot]
            h_chunk = [h_v[slot, pl.ds(16 * c, 16)] for c in range(4)]

            @pl.loop(0, N_GROUPS)
            def per_group(g):
                s0 = jnp.where(g < N_GROUPS - 1, g * 16, TAIL_OUT)
                ts = []
                for j in range(16):
                    l = [rows[s0 + j, pl.ds(16 * c, 16)] for c in range(4)]
                    ts.append((l[0] * h_chunk[0] + l[1] * h_chunk[1])
                              + (l[2] * h_chunk[2] + l[3] * h_chunk[3]))
                for s in (1, 2, 4, 8):
                    nxt = []
                    for a in range(0, len(ts), 2):
                        lo = ts[a] + shuffle(ts[a], s)
                        hi = ts[a + 1] + shuffle(ts[a + 1], s)
                        nxt.append(jnp.where(odd[s], hi, lo))
                    ts = nxt
                out_v[slot, pl.ds(s0, 16)] = ts[0]

        def step(k, slot):
            @pl.when(k < n_per_worker - 2)
            def _():
                s2 = (slot + 2) % NBUF
                drain(idx_hbm.at[base_b], idx_v.at[s2], idx_sems.at[s2])
                issue_rows(k + 2, s2)
                issue_h(k + 2, s2)

            @pl.when(k >= NBUF)
            def _():
                drain(out_hbm.at[base_b, 0], out_v.at[slot],
                      out_sems.at[slot])

            drain(w_hbm.at[pl.ds(0, SAMPLE)], rows_v.at[slot],
                  rows_sems.at[slot])
            drain(h_hbm.at[base_b, 0], h_v.at[slot], h_sems.at[slot])

            @pl.when(k < n_per_worker - NBUF)
            def _():
                issue_idx(k + NBUF, slot)

            compute(slot)

            pltpu.async_copy(out_v.at[slot], out_hbm.at[base_b + k, 0],
                             out_sems.at[slot])

        for i in range(NBUF):
            issue_idx(i, i)
        for i in range(2):
            drain(idx_hbm.at[base_b], idx_v.at[i], idx_sems.at[i])
            issue_rows(i, i)
            issue_h(i, i)

        @pl.loop(0, n_per_worker, step=2)
        def per_pair(k):
            step(k, (k % NBUF))
            step(k + 1, (k + 1) % NBUF)

        for i in range(NBUF):
            drain(out_hbm.at[base_b, 0], out_v.at[i], out_sems.at[i])

    return emb_dot


@jax.jit
def kernel(h, indicies, W):
    batch = h.shape[0]
    n_workers = 32
    idx3 = jnp.reshape(indicies.astype(jnp.int32),
                       (batch, N_CHUNKS, GATHER_CHUNK))
    return _make_kernel(batch, batch // n_workers)(h, idx3, W)


# speedup vs baseline: 1.0588x; 1.0588x over previous
import functools

import jax
import jax.numpy as jnp
from jax import lax
from jax.experimental import pallas as pl
from jax.experimental.pallas import tpu as pltpu
from jax.experimental.pallas import tpu_sc as plsc

D_MODEL = 64
ROW_PITCH = 128
SAMPLE = 200
GATHER_CHUNK = 104
CHUNK_OFFS = (0, SAMPLE - GATHER_CHUNK)
ROWS_BUF = 208
N_GROUPS = 13
TAIL_OUT = SAMPLE - 16
NBUF = 4


def _make_kernel(batch, n_per_worker):
    mesh = plsc.VectorSubcoreMesh(core_axis_name="c", subcore_axis_name="s")
    num_cores = 2

    @functools.partial(
        pl.kernel,
        out_type=jax.ShapeDtypeStruct((batch * SAMPLE,), jnp.float32),
        mesh=mesh,
        compiler_params=pltpu.CompilerParams(
            needs_layout_passes=False, use_tc_tiling_on_sc=False),
        scratch_types=[
            pltpu.VMEM((NBUF * ROWS_BUF,), jnp.int32),
            pltpu.VMEM((NBUF * ROWS_BUF, D_MODEL), jnp.float32),
            pltpu.VMEM((NBUF * D_MODEL,), jnp.float32),
            pltpu.VMEM((NBUF * SAMPLE,), jnp.float32),
            pltpu.SemaphoreType.DMA((NBUF,)),
            pltpu.SemaphoreType.DMA((NBUF,)),
            pltpu.SemaphoreType.DMA((NBUF,)),
            pltpu.SemaphoreType.DMA((NBUF,)),
        ],
    )
    def emb_dot(h_hbm, idx_hbm, w_hbm, out_hbm, idx_v, rows_v, h_v, out_v,
                idx_sems, rows_sems, h_sems, out_sems):
        wid = lax.axis_index("s") * num_cores + lax.axis_index("c")
        base_b = wid * n_per_worker

        lane = lax.iota(jnp.int32, 16)
        perm = {s: lane ^ s for s in (1, 2, 4, 8)}
        odd = {s: (lane & s) != 0 for s in (1, 2, 4, 8)}

        def idx_copy(k, slot):
            return pltpu.make_async_copy(
                idx_hbm.at[pl.ds((base_b + k) * SAMPLE, SAMPLE)],
                idx_v.at[pl.ds(slot * ROWS_BUF, SAMPLE)],
                idx_sems.at[slot])

        def rows_copies(k, slot):
            return [
                pltpu.make_async_copy(
                    w_hbm.at[idx_v.at[pl.ds(slot * ROWS_BUF + off,
                                            GATHER_CHUNK)]],
                    rows_v.at[pl.ds(slot * ROWS_BUF + j * GATHER_CHUNK,
                                    GATHER_CHUNK), :],
                    rows_sems.at[slot])
                for j, off in enumerate(CHUNK_OFFS)
            ]

        def h_copy(k, slot):
            return pltpu.make_async_copy(
                h_hbm.at[pl.ds((base_b + k) * D_MODEL, D_MODEL)],
                h_v.at[pl.ds(slot * D_MODEL, D_MODEL)],
                h_sems.at[slot])

        def out_copy(k, slot):
            return pltpu.make_async_copy(
                out_v.at[pl.ds(slot * SAMPLE, SAMPLE)],
                out_hbm.at[pl.ds((base_b + k) * SAMPLE, SAMPLE)],
                out_sems.at[slot])

        def shuffle(v, s):
            return jnp.take(v, perm[s], axis=0, unique_indices=True)

        def compute(slot):
            row0 = slot * ROWS_BUF
            h0 = slot * D_MODEL
            h_chunk = [h_v[pl.ds(h0 + 16 * c, 16)] for c in range(4)]

            @pl.loop(0, N_GROUPS)
            def per_group(g):
                os = jnp.where(g < N_GROUPS - 1, g * 16, TAIL_OUT)
                rs = row0 + os + jnp.where(os >= 96, 8, 0)
                ts = []
                for j in range(16):
                    l = [rows_v[rs + j, pl.ds(16 * c, 16)] for c in range(4)]
                    ts.append((l[0] * h_chunk[0] + l[1] * h_chunk[1])
                              + (l[2] * h_chunk[2] + l[3] * h_chunk[3]))
                for s in (1, 2, 4, 8):
                    nxt = []
                    for a in range(0, len(ts), 2):
                        lo = ts[a] + shuffle(ts[a], s)
                        hi = ts[a + 1] + shuffle(ts[a + 1], s)
                        nxt.append(jnp.where(odd[s], hi, lo))
                    ts = nxt
                out_v[pl.ds(slot * SAMPLE + os, 16)] = ts[0]

        def step(k, slot):
            @pl.when(k < n_per_worker - 2)
            def _():
                s2 = (slot + 2) % NBUF
                idx_copy(k + 2, s2).wait()
                for c in rows_copies(k + 2, s2):
                    c.start()
                h_copy(k + 2, s2).start()

            @pl.when(k >= NBUF)
            def _():
                out_copy(k, slot).wait()

            for c in rows_copies(k, slot):
                c.wait()
            h_copy(k, slot).wait()

            @pl.when(k < n_per_worker - NBUF)
            def _():
                idx_copy(k + NBUF, slot).start()

            compute(slot)

            out_copy(k, slot).start()

        for i in range(NBUF):
            idx_copy(i, i).start()
        for i in range(2):
            idx_copy(i, i).wait()
            for c in rows_copies(i, i):
                c.start()
            h_copy(i, i).start()

        @pl.loop(0, n_per_worker, step=2)
        def per_pair(k):
            step(k, (k % NBUF))
            step(k + 1, (k + 1) % NBUF)

        for i in range(NBUF):
            out_copy(n_per_worker - NBUF + i, i).wait()

    return emb_dot


@jax.jit
def kernel(h, indicies, W):
    batch = h.shape[0]
    n_workers = 32
    h1 = jnp.reshape(h, (-1,))
    idx1 = jnp.reshape(indicies.astype(jnp.int32), (-1,))
    out = _make_kernel(batch, batch // n_workers)(h1, idx1, W)
    return jnp.reshape(out, (batch, 1, SAMPLE))
